# baseline (speedup 1.0000x reference)
import functools

import jax
import jax.numpy as jnp
from jax import lax
from jax.experimental import pallas as pl
from jax.experimental.pallas import tpu as pltpu
from jax.experimental.pallas import tpu_sc as plsc

NC = 2
NS = 16
L = 16
NW = NC * NS
C = 80


def _sc_scores(x_email, x_noun, i0, i1, per, nchunk):
    total = per * NW
    d_model = x_email.shape[1]
    mesh = plsc.VectorSubcoreMesh(core_axis_name="c", subcore_axis_name="s")

    @functools.partial(
        pl.kernel,
        mesh=mesh,
        compiler_params=pltpu.CompilerParams(use_tc_tiling_on_sc=False,
                                             needs_layout_passes=False),
        out_type=jax.ShapeDtypeStruct((total,), jnp.float32),
        scratch_types=[
            pltpu.VMEM((per,), jnp.int32),
            pltpu.VMEM((per,), jnp.int32),
            pltpu.VMEM((per,), jnp.float32),
            pltpu.VMEM((C, d_model), jnp.float32),
            pltpu.VMEM((C, d_model), jnp.float32),
            pltpu.VMEM((C, d_model), jnp.float32),
            pltpu.VMEM((C, d_model), jnp.float32),
            pltpu.SemaphoreType.DMA,
            pltpu.SemaphoreType.DMA,
            pltpu.SemaphoreType.DMA,
            pltpu.SemaphoreType.DMA,
        ],
    )
    def k(xe_hbm, xn_hbm, i0_hbm, i1_hbm, out_hbm,
          i0_v, i1_v, out_v, a0, b0, a1, b1, sa0, sb0, sa1, sb1):
        wid = lax.axis_index("s") * NC + lax.axis_index("c")
        base = wid * per
        pltpu.sync_copy(i0_hbm.at[pl.ds(base, per)], i0_v)
        pltpu.sync_copy(i1_hbm.at[pl.ds(base, per)], i1_v)
        lane = lax.iota(jnp.int32, L)
        zero = jnp.zeros((L,), jnp.float32)

        def copies(it, buf_a, buf_b, sem_a, sem_b):
            off = it * C
            return (
                pltpu.make_async_copy(
                    xe_hbm.at[i0_v.at[pl.ds(off, C)]], buf_a, sem_a),
                pltpu.make_async_copy(
                    xn_hbm.at[i1_v.at[pl.ds(off, C)]], buf_b, sem_b),
            )

        def start(it, buf_a, buf_b, sem_a, sem_b):
            ca, cb = copies(it, buf_a, buf_b, sem_a, sem_b)
            ca.start()
            cb.start()

        def compute(it, buf_a, buf_b, sem_a, sem_b):
            ca, cb = copies(it, buf_a, buf_b, sem_a, sem_b)
            ca.wait()
            cb.wait()
            off = it * C
            for g in range(C // L):
                rows = lane + g * L

                def dbody(d, accs):
                    res = []
                    for j in range(4):
                        cols = jnp.full((L,), d + j, jnp.int32)
                        a = plsc.load_gather(buf_a, [rows, cols])
                        b = plsc.load_gather(buf_b, [rows, cols])
                        res.append(accs[j] + a * b)
                    return tuple(res)

                accs = plsc.parallel_loop(
                    0, d_model, step=4, unroll=4,
                    carry=(zero, zero, zero, zero))(dbody)
                out_v[pl.ds(off + g * L, L)] = (
                    (accs[0] + accs[1]) + (accs[2] + accs[3]))

        start(0, a0, b0, sa0, sb0)
        npair = nchunk // 2

        def pair(p, carry):
            it0 = 2 * p
            start(it0 + 1, a1, b1, sa1, sb1)
            compute(it0, a0, b0, sa0, sb0)

            @pl.when(it0 + 2 < nchunk)
            def _():
                start(it0 + 2, a0, b0, sa0, sb0)

            compute(it0 + 1, a1, b1, sa1, sb1)
            return carry

        lax.fori_loop(0, npair, pair, 0)
        pltpu.sync_copy(out_v, out_hbm.at[pl.ds(base, per)])

    return k(x_email, x_noun, i0, i1)


def kernel(x_email, x_noun, edge_label_index):
    n_edges = edge_label_index.shape[1]
    per = -(-n_edges // (NW * 2 * C)) * (2 * C)
    total = per * NW
    idx = edge_label_index.astype(jnp.int32)
    i0 = jnp.pad(idx[0], (0, total - n_edges))
    i1 = jnp.pad(idx[1], (0, total - n_edges))
    out = _sc_scores(x_email, x_noun, i0, i1, per, per // C)
    return out[:n_edges]


# speedup vs baseline: 3.1959x; 3.1959x over previous
import functools

import jax
import jax.numpy as jnp
from jax import lax
from jax.experimental import pallas as pl
from jax.experimental.pallas import tpu as pltpu
from jax.experimental.pallas import tpu_sc as plsc

NC = 2
NS = 16
L = 16
NW = NC * NS
C = 80


def _sc_scores(x_email, x_noun, i0, i1, per, nchunk):
    total = per * NW
    d_model = x_email.shape[1]
    mesh = plsc.VectorSubcoreMesh(core_axis_name="c", subcore_axis_name="s")

    @functools.partial(
        pl.kernel,
        mesh=mesh,
        compiler_params=pltpu.CompilerParams(use_tc_tiling_on_sc=False,
                                             needs_layout_passes=False),
        out_type=jax.ShapeDtypeStruct((total,), jnp.float32),
        scratch_types=[
            pltpu.VMEM((per,), jnp.int32),
            pltpu.VMEM((per,), jnp.int32),
            pltpu.VMEM((per,), jnp.float32),
            pltpu.VMEM((C, d_model), jnp.float32),
            pltpu.VMEM((C, d_model), jnp.float32),
            pltpu.VMEM((C, d_model), jnp.float32),
            pltpu.VMEM((C, d_model), jnp.float32),
            pltpu.VMEM((L, L + 1), jnp.float32),
            pltpu.SemaphoreType.DMA,
            pltpu.SemaphoreType.DMA,
            pltpu.SemaphoreType.DMA,
            pltpu.SemaphoreType.DMA,
        ],
    )
    def k(xe_hbm, xn_hbm, i0_hbm, i1_hbm, out_hbm,
          i0_v, i1_v, out_v, a0, b0, a1, b1, s_ref, sa0, sb0, sa1, sb1):
        wid = lax.axis_index("s") * NC + lax.axis_index("c")
        base = wid * per
        pltpu.sync_copy(i0_hbm.at[pl.ds(base, per)], i0_v)
        pltpu.sync_copy(i1_hbm.at[pl.ds(base, per)], i1_v)
        lane = lax.iota(jnp.int32, L)
        zero = jnp.zeros((L,), jnp.float32)

        def copies(it, buf_a, buf_b, sem_a, sem_b):
            off = it * C
            return (
                pltpu.make_async_copy(
                    xe_hbm.at[i0_v.at[pl.ds(off, C)]], buf_a, sem_a),
                pltpu.make_async_copy(
                    xn_hbm.at[i1_v.at[pl.ds(off, C)]], buf_b, sem_b),
            )

        def start(it, buf_a, buf_b, sem_a, sem_b):
            ca, cb = copies(it, buf_a, buf_b, sem_a, sem_b)
            ca.start()
            cb.start()

        def compute(it, buf_a, buf_b, sem_a, sem_b):
            ca, cb = copies(it, buf_a, buf_b, sem_a, sem_b)
            ca.wait()
            cb.wait()
            off = it * C

            def group(g, carry):
                e0 = g * L
                for l in range(L):
                    e = e0 + l
                    acc = (buf_a[e, pl.ds(0, L)] * buf_b[e, pl.ds(0, L)])
                    for j in range(1, d_model // L):
                        acc = acc + (buf_a[e, pl.ds(j * L, L)]
                                     * buf_b[e, pl.ds(j * L, L)])
                    s_ref[l, pl.ds(0, L)] = acc
                tot = zero
                for k in range(L):
                    cols = jnp.full((L,), k, jnp.int32)
                    tot = tot + plsc.load_gather(s_ref, [lane, cols])
                out_v[pl.ds(off + e0, L)] = tot
                return carry

            lax.fori_loop(0, C // L, group, 0)

        start(0, a0, b0, sa0, sb0)
        npair = nchunk // 2

        def pair(p, carry):
            it0 = 2 * p
            start(it0 + 1, a1, b1, sa1, sb1)
            compute(it0, a0, b0, sa0, sb0)

            @pl.when(it0 + 2 < nchunk)
            def _():
                start(it0 + 2, a0, b0, sa0, sb0)

            compute(it0 + 1, a1, b1, sa1, sb1)
            return carry

        lax.fori_loop(0, npair, pair, 0)
        pltpu.sync_copy(out_v, out_hbm.at[pl.ds(base, per)])

    return k(x_email, x_noun, i0, i1)


def kernel(x_email, x_noun, edge_label_index):
    n_edges = edge_label_index.shape[1]
    per = -(-n_edges // (NW * 2 * C)) * (2 * C)
    total = per * NW
    idx = edge_label_index.astype(jnp.int32)
    i0 = jnp.pad(idx[0], (0, total - n_edges))
    i1 = jnp.pad(idx[1], (0, total - n_edges))
    out = _sc_scores(x_email, x_noun, i0, i1, per, per // C)
    return out[:n_edges]


# speedup vs baseline: 4.1099x; 1.2860x over previous
import functools

import jax
import jax.numpy as jnp
from jax import lax
from jax.experimental import pallas as pl
from jax.experimental.pallas import tpu as pltpu
from jax.experimental.pallas import tpu_sc as plsc

NC = 2
NS = 16
L = 16
NW = NC * NS
C = 80


def _sc_scores(x_email, x_noun, i0, i1, per, nchunk):
    total = per * NW
    d_model = x_email.shape[1]
    mesh = plsc.VectorSubcoreMesh(core_axis_name="c", subcore_axis_name="s")

    @functools.partial(
        pl.kernel,
        mesh=mesh,
        compiler_params=pltpu.CompilerParams(use_tc_tiling_on_sc=False,
                                             needs_layout_passes=False),
        out_type=jax.ShapeDtypeStruct((total,), jnp.float32),
        scratch_types=[
            pltpu.VMEM((per,), jnp.int32),
            pltpu.VMEM((per,), jnp.int32),
            pltpu.VMEM((per,), jnp.float32),
            pltpu.VMEM((C, d_model), jnp.bfloat16),
            pltpu.VMEM((C, d_model), jnp.bfloat16),
            pltpu.VMEM((C, d_model), jnp.bfloat16),
            pltpu.VMEM((C, d_model), jnp.bfloat16),
            pltpu.VMEM((L, L + 1), jnp.float32),
            pltpu.SemaphoreType.DMA,
            pltpu.SemaphoreType.DMA,
            pltpu.SemaphoreType.DMA,
            pltpu.SemaphoreType.DMA,
        ],
    )
    def k(xe_hbm, xn_hbm, i0_hbm, i1_hbm, out_hbm,
          i0_v, i1_v, out_v, a0, b0, a1, b1, s_ref, sa0, sb0, sa1, sb1):
        wid = lax.axis_index("s") * NC + lax.axis_index("c")
        base = wid * per
        pltpu.sync_copy(i0_hbm.at[pl.ds(base, per)], i0_v)
        pltpu.sync_copy(i1_hbm.at[pl.ds(base, per)], i1_v)
        lane = lax.iota(jnp.int32, L)
        zero = jnp.zeros((L,), jnp.float32)

        def copies(it, buf_a, buf_b, sem_a, sem_b):
            off = it * C
            return (
                pltpu.make_async_copy(
                    xe_hbm.at[i0_v.at[pl.ds(off, C)]], buf_a, sem_a),
                pltpu.make_async_copy(
                    xn_hbm.at[i1_v.at[pl.ds(off, C)]], buf_b, sem_b),
            )

        def start(it, buf_a, buf_b, sem_a, sem_b):
            ca, cb = copies(it, buf_a, buf_b, sem_a, sem_b)
            ca.start()
            cb.start()

        def compute(it, buf_a, buf_b, sem_a, sem_b):
            ca, cb = copies(it, buf_a, buf_b, sem_a, sem_b)
            ca.wait()
            cb.wait()
            off = it * C

            def group(g, carry):
                e0 = g * L
                for l in range(L):
                    e = e0 + l
                    acc = zero
                    for j in range(d_model // (2 * L)):
                        ra = buf_a[e, pl.ds(j * 2 * L, 2 * L)]
                        rb = buf_b[e, pl.ds(j * 2 * L, 2 * L)]
                        a_ev, a_od = plsc.unpack(
                            ra, format=plsc.PackFormat.INTERLEAVED,
                            preferred_element_type=jnp.float32)
                        b_ev, b_od = plsc.unpack(
                            rb, format=plsc.PackFormat.INTERLEAVED,
                            preferred_element_type=jnp.float32)
                        acc = acc + (a_ev * b_ev + a_od * b_od)
                    s_ref[l, pl.ds(0, L)] = acc
                tot = zero
                for k in range(L):
                    cols = jnp.full((L,), k, jnp.int32)
                    tot = tot + plsc.load_gather(s_ref, [lane, cols])
                out_v[pl.ds(off + e0, L)] = tot
                return carry

            lax.fori_loop(0, C // L, group, 0)

        start(0, a0, b0, sa0, sb0)
        npair = nchunk // 2

        def pair(p, carry):
            it0 = 2 * p
            start(it0 + 1, a1, b1, sa1, sb1)
            compute(it0, a0, b0, sa0, sb0)

            @pl.when(it0 + 2 < nchunk)
            def _():
                start(it0 + 2, a0, b0, sa0, sb0)

            compute(it0 + 1, a1, b1, sa1, sb1)
            return carry

        lax.fori_loop(0, npair, pair, 0)
        pltpu.sync_copy(out_v, out_hbm.at[pl.ds(base, per)])

    return k(x_email, x_noun, i0, i1)


def kernel(x_email, x_noun, edge_label_index):
    n_edges = edge_label_index.shape[1]
    x_email = x_email.astype(jnp.bfloat16)
    x_noun = x_noun.astype(jnp.bfloat16)
    per = -(-n_edges // (NW * 2 * C)) * (2 * C)
    total = per * NW
    idx = edge_label_index.astype(jnp.int32)
    i0 = jnp.pad(idx[0], (0, total - n_edges))
    i1 = jnp.pad(idx[1], (0, total - n_edges))
    out = _sc_scores(x_email, x_noun, i0, i1, per, per // C)
    return out[:n_edges]


# speedup vs baseline: 5.8211x; 1.4164x over previous
import functools

import jax
import jax.numpy as jnp
from jax import lax
from jax.experimental import pallas as pl
from jax.experimental.pallas import tpu as pltpu
from jax.experimental.pallas import tpu_sc as plsc

NC = 2
NS = 16
L = 16
NW = NC * NS
C = 48


def _sc_scores(x_email, x_noun, eli, per_real, per, nchunk):
    total = per_real * NW
    d_model = x_email.shape[1]
    mesh = plsc.VectorSubcoreMesh(core_axis_name="c", subcore_axis_name="s")

    @functools.partial(
        pl.kernel,
        mesh=mesh,
        compiler_params=pltpu.CompilerParams(use_tc_tiling_on_sc=False,
                                             needs_layout_passes=False),
        out_type=jax.ShapeDtypeStruct((total,), jnp.float32),
        scratch_types=[
            pltpu.VMEM((per,), jnp.int32),
            pltpu.VMEM((per,), jnp.int32),
            pltpu.VMEM((per,), jnp.float32),
            pltpu.VMEM((C, d_model), jnp.bfloat16),
            pltpu.VMEM((C, d_model), jnp.bfloat16),
            pltpu.VMEM((C, d_model), jnp.bfloat16),
            pltpu.VMEM((C, d_model), jnp.bfloat16),
            pltpu.VMEM((L, L + 1), jnp.float32),
            pltpu.VMEM_SHARED((10000, 256), jnp.bfloat16),
            pltpu.SemaphoreType.DMA,
            pltpu.SemaphoreType.DMA,
            pltpu.SemaphoreType.DMA,
            pltpu.SemaphoreType.DMA,
        ],
    )
    def k(xe_hbm, xn_hbm, eli_hbm, out_hbm,
          i0_v, i1_v, out_v, a0, b0, a1, b1, s_ref, xe_sp,
          sa0, sb0, sa1, sb1):
        wid = lax.axis_index("s") * NC + lax.axis_index("c")
        base = wid * per_real
        sid = lax.axis_index("s")
        rows_per_tile = xe_hbm.shape[0] // NS
        pltpu.sync_copy(
            xe_hbm.at[pl.ds(sid * rows_per_tile, rows_per_tile)],
            xe_sp.at[pl.ds(sid * rows_per_tile, rows_per_tile)])
        pltpu.sync_copy(eli_hbm.at[0, pl.ds(base, per_real)],
                        i0_v.at[pl.ds(0, per_real)])
        pltpu.sync_copy(eli_hbm.at[1, pl.ds(base, per_real)],
                        i1_v.at[pl.ds(0, per_real)])
        lane = lax.iota(jnp.int32, L)
        zero = jnp.zeros((L,), jnp.float32)
        zero_i = jnp.zeros((L,), jnp.int32)
        for pad_off in range(per_real, per, L):
            o = min(pad_off, per - L)
            i0_v[pl.ds(o, L)] = zero_i
            i1_v[pl.ds(o, L)] = zero_i

        def copies(it, buf_a, buf_b, sem_a, sem_b):
            off = it * C
            return (
                pltpu.make_async_copy(
                    xe_sp.at[i0_v.at[pl.ds(off, C)]], buf_a, sem_a),
                pltpu.make_async_copy(
                    xn_hbm.at[i1_v.at[pl.ds(off, C)]], buf_b, sem_b),
            )

        def start(it, buf_a, buf_b, sem_a, sem_b):
            ca, cb = copies(it, buf_a, buf_b, sem_a, sem_b)
            ca.start()
            cb.start()

        def compute(it, buf_a, buf_b, sem_a, sem_b):
            ca, cb = copies(it, buf_a, buf_b, sem_a, sem_b)
            ca.wait()
            cb.wait()
            off = it * C

            def group(g, carry):
                e0 = g * L
                for l in range(L):
                    e = e0 + l
                    acc = zero
                    for j in range(d_model // (2 * L)):
                        ra = buf_a[e, pl.ds(j * 2 * L, 2 * L)]
                        rb = buf_b[e, pl.ds(j * 2 * L, 2 * L)]
                        a_ev, a_od = plsc.unpack(
                            ra, format=plsc.PackFormat.INTERLEAVED,
                            preferred_element_type=jnp.float32)
                        b_ev, b_od = plsc.unpack(
                            rb, format=plsc.PackFormat.INTERLEAVED,
                            preferred_element_type=jnp.float32)
                        acc = acc + (a_ev * b_ev + a_od * b_od)
                    s_ref[l, pl.ds(0, L)] = acc
                tot = zero
                for k in range(L):
                    cols = jnp.full((L,), k, jnp.int32)
                    tot = tot + plsc.load_gather(s_ref, [lane, cols])
                out_v[pl.ds(off + e0, L)] = tot
                return carry

            lax.fori_loop(0, C // L, group, 0)

        plsc.subcore_barrier()
        start(0, a0, b0, sa0, sb0)
        npair = nchunk // 2

        def pair(p, carry):
            it0 = 2 * p
            start(it0 + 1, a1, b1, sa1, sb1)
            compute(it0, a0, b0, sa0, sb0)

            @pl.when(it0 + 2 < nchunk)
            def _():
                start(it0 + 2, a0, b0, sa0, sb0)

            compute(it0 + 1, a1, b1, sa1, sb1)
            return carry

        lax.fori_loop(0, npair, pair, 0)
        if nchunk % 2:
            compute(nchunk - 1, a0, b0, sa0, sb0)
        pltpu.sync_copy(out_v.at[pl.ds(0, per_real)],
                        out_hbm.at[pl.ds(base, per_real)])

    return k(x_email, x_noun, eli)


def kernel(x_email, x_noun, edge_label_index):
    n_edges = edge_label_index.shape[1]
    x_email = x_email.astype(jnp.bfloat16)
    x_noun = x_noun.astype(jnp.bfloat16)
    per_real = n_edges // NW
    per = -(-per_real // C) * C
    eli = edge_label_index.astype(jnp.int32)
    return _sc_scores(x_email, x_noun, eli, per_real, per, per // C)


# speedup vs baseline: 6.5630x; 1.1275x over previous
import functools

import jax
import jax.numpy as jnp
from jax import lax
from jax.experimental import pallas as pl
from jax.experimental.pallas import tpu as pltpu
from jax.experimental.pallas import tpu_sc as plsc

NC = 2
NS = 16
L = 16
NW = NC * NS
C = 48


def _sc_scores(x_email, x_noun, eli, per_real, per, nchunk):
    total = per_real * NW
    d_model = x_email.shape[1]
    mesh = plsc.VectorSubcoreMesh(core_axis_name="c", subcore_axis_name="s")

    @functools.partial(
        pl.kernel,
        mesh=mesh,
        compiler_params=pltpu.CompilerParams(use_tc_tiling_on_sc=False,
                                             needs_layout_passes=False),
        out_type=jax.ShapeDtypeStruct((total,), jnp.float32),
        scratch_types=[
            pltpu.VMEM((per,), jnp.int32),
            pltpu.VMEM((per,), jnp.int32),
            pltpu.VMEM((per,), jnp.float32),
            pltpu.VMEM((C, d_model), jnp.bfloat16),
            pltpu.VMEM((C, d_model), jnp.bfloat16),
            pltpu.VMEM((C, d_model), jnp.bfloat16),
            pltpu.VMEM((C, d_model), jnp.bfloat16),
            pltpu.VMEM((L, L + 1), jnp.float32),
            pltpu.VMEM_SHARED((10000, 256), jnp.bfloat16),
            pltpu.SemaphoreType.DMA,
            pltpu.SemaphoreType.DMA,
            pltpu.SemaphoreType.DMA,
            pltpu.SemaphoreType.DMA,
        ],
    )
    def k(xe_hbm, xn_hbm, eli_hbm, out_hbm,
          i0_v, i1_v, out_v, a0, b0, a1, b1, s_ref, xe_sp,
          sa0, sb0, sa1, sb1):
        wid = lax.axis_index("s") * NC + lax.axis_index("c")
        base = wid * per_real
        sid = lax.axis_index("s")
        rows_per_tile = xe_hbm.shape[0] // NS
        pltpu.sync_copy(
            xe_hbm.at[pl.ds(sid * rows_per_tile, rows_per_tile)],
            xe_sp.at[pl.ds(sid * rows_per_tile, rows_per_tile)])
        pltpu.sync_copy(eli_hbm.at[0, pl.ds(base, per_real)],
                        i0_v.at[pl.ds(0, per_real)])
        pltpu.sync_copy(eli_hbm.at[1, pl.ds(base, per_real)],
                        i1_v.at[pl.ds(0, per_real)])
        lane = lax.iota(jnp.int32, L)
        zero = jnp.zeros((L,), jnp.float32)
        zero_i = jnp.zeros((L,), jnp.int32)
        for pad_off in range(per_real, per, L):
            o = min(pad_off, per - L)
            i0_v[pl.ds(o, L)] = zero_i
            i1_v[pl.ds(o, L)] = zero_i

        def copies(it, buf_a, buf_b, sem_a, sem_b):
            off = it * C
            return (
                pltpu.make_async_copy(
                    xe_sp.at[i0_v.at[pl.ds(off, C)]], buf_a, sem_a),
                pltpu.make_async_copy(
                    xn_hbm.at[i1_v.at[pl.ds(off, C)]], buf_b, sem_b),
            )

        def start(it, buf_a, buf_b, sem_a, sem_b):
            ca, cb = copies(it, buf_a, buf_b, sem_a, sem_b)
            ca.start()
            cb.start()

        def compute(it, buf_a, buf_b, sem_a, sem_b):
            ca, cb = copies(it, buf_a, buf_b, sem_a, sem_b)
            ca.wait()
            cb.wait()
            off = it * C

            def group(g, carry):
                e0 = g * L
                for l in range(L):
                    e = e0 + l
                    acc = zero
                    for j in range(d_model // (2 * L)):
                        ra = buf_a[e, pl.ds(j * 2 * L, 2 * L)]
                        rb = buf_b[e, pl.ds(j * 2 * L, 2 * L)]
                        p_ev, p_od = plsc.unpack(
                            ra * rb, format=plsc.PackFormat.INTERLEAVED,
                            preferred_element_type=jnp.float32)
                        acc = acc + (p_ev + p_od)
                    s_ref[l, pl.ds(0, L)] = acc
                tot = zero
                for k in range(L):
                    cols = jnp.full((L,), k, jnp.int32)
                    tot = tot + plsc.load_gather(s_ref, [lane, cols])
                out_v[pl.ds(off + e0, L)] = tot
                return carry

            lax.fori_loop(0, C // L, group, 0)

        plsc.subcore_barrier()
        start(0, a0, b0, sa0, sb0)
        npair = nchunk // 2

        def pair(p, carry):
            it0 = 2 * p
            start(it0 + 1, a1, b1, sa1, sb1)
            compute(it0, a0, b0, sa0, sb0)

            @pl.when(it0 + 2 < nchunk)
            def _():
                start(it0 + 2, a0, b0, sa0, sb0)

            compute(it0 + 1, a1, b1, sa1, sb1)
            return carry

        lax.fori_loop(0, npair, pair, 0)
        if nchunk % 2:
            compute(nchunk - 1, a0, b0, sa0, sb0)
        pltpu.sync_copy(out_v.at[pl.ds(0, per_real)],
                        out_hbm.at[pl.ds(base, per_real)])

    return k(x_email, x_noun, eli)


def kernel(x_email, x_noun, edge_label_index):
    n_edges = edge_label_index.shape[1]
    x_email = x_email.astype(jnp.bfloat16)
    x_noun = x_noun.astype(jnp.bfloat16)
    per_real = n_edges // NW
    per = -(-per_real // C) * C
    eli = edge_label_index.astype(jnp.int32)
    return _sc_scores(x_email, x_noun, eli, per_real, per, per // C)
